# baseline (speedup 1.0000x reference)
import functools

import jax
import jax.numpy as jnp
from jax import lax
from jax.experimental import pallas as pl
from jax.experimental.pallas import tpu as pltpu
from jax.experimental.pallas import tpu_sc as plsc

_B = 16384
_H = 64


def _make_gather_kernel():
    info = plsc.get_sparse_core_info()
    nc, ns = info.num_cores, info.num_subcores
    nw = nc * ns
    bpw = _B // nw

    mesh = plsc.VectorSubcoreMesh(core_axis_name="c", subcore_axis_name="s")

    @functools.partial(
        pl.kernel,
        mesh=mesh,
        out_type=[
            jax.ShapeDtypeStruct((_B, 2 * _H), jnp.float32),
            jax.ShapeDtypeStruct((_B, 2 * _H), jnp.float32),
        ],
        scratch_types=[
            pltpu.VMEM((bpw,), jnp.int32),
            pltpu.VMEM((bpw,), jnp.int32),
            pltpu.VMEM((128, 2 * _H), jnp.float32),
            pltpu.VMEM((128, 2 * _H), jnp.float32),
            pltpu.VMEM((128, 2 * _H), jnp.float32),
            pltpu.VMEM((128, 2 * _H), jnp.float32),
            pltpu.SemaphoreType.DMA,
            pltpu.SemaphoreType.DMA,
            pltpu.SemaphoreType.DMA,
            pltpu.SemaphoreType.DMA,
            pltpu.SemaphoreType.DMA,
            pltpu.SemaphoreType.DMA,
            pltpu.SemaphoreType.DMA,
            pltpu.SemaphoreType.DMA,
        ],
    )
    def gk(mem_hbm, s_hbm, d_hbm, es_hbm, ed_hbm,
           sidx_v, didx_v, buf0, buf1, buf2, buf3,
           g0, g1, g2, g3, w0, w1, w2, w3):
        wid = lax.axis_index("s") * nc + lax.axis_index("c")
        base = wid * bpw
        ch = 128
        bufs = (buf0, buf1, buf2, buf3)
        gsems = (g0, g1, g2, g3)
        wsems = (w0, w1, w2, w3)

        pltpu.sync_copy(s_hbm.at[pl.ds(base, bpw)], sidx_v)
        pltpu.sync_copy(d_hbm.at[pl.ds(base, bpw)], didx_v)

        def fire_chunk(idx_ref, off, buf, sem):
            def fire(g, carry):
                j0 = g * 16
                v = idx_ref[pl.ds(off + j0, 16)]
                tv = lax.shift_right_logical(v, 1)
                for lane in range(16):
                    pltpu.make_async_copy(
                        mem_hbm.at[pl.ds(tv[lane], 1), :],
                        buf.at[pl.ds(j0 + lane, 1), :], sem).start()
                return carry
            lax.fori_loop(0, ch // 16, fire, 0)

        nch = bpw // ch
        writes = [None] * 4
        for c in range(2 * nch):
            b = c % 4
            if c >= 4:
                writes[b].wait()
            if c < nch:
                idx_ref, off, dst = sidx_v, c * ch, es_hbm
            else:
                idx_ref, off, dst = didx_v, (c - nch) * ch, ed_hbm
            fire_chunk(idx_ref, off, bufs[b], gsems[b])
            pltpu.make_async_copy(
                es_hbm.at[pl.ds(0, ch)], bufs[b], gsems[b]).wait()
            writes[b] = pltpu.async_copy(
                bufs[b], dst.at[pl.ds(base + off, ch)], wsems[b])
        for b in range(4):
            writes[b].wait()

    return gk


_gather = _make_gather_kernel()


def _mlp_body(es_ref, ed_ref, sp_ref, dp_ref, w1a_ref, w1b_ref, b1_ref,
              w2_ref, b2_ref, out_ref):
    xs = es_ref[...]
    xd = ed_ref[...]
    se = jnp.dot(xs[:, :_H], w1a_ref[...], preferred_element_type=jnp.float32)
    so = jnp.dot(xs[:, _H:], w1a_ref[...], preferred_element_type=jnp.float32)
    de = jnp.dot(xd[:, :_H], w1b_ref[...], preferred_element_type=jnp.float32)
    do = jnp.dot(xd[:, _H:], w1b_ref[...], preferred_element_type=jnp.float32)
    x = (jnp.where(sp_ref[...] == 0, se, so)
         + jnp.where(dp_ref[...] == 0, de, do)
         + b1_ref[...])
    h = jnp.maximum(x, 0.0)
    o = jnp.sum(h * w2_ref[...], axis=1) + b2_ref[0, 0]
    out_ref[...] = o.reshape(1, 1, -1)


def _mlp_tc(es, ed, sp, dp, w1a, w1b, b1r, w2r, b2s):
    blk = 2048
    g = _B // blk
    out = pl.pallas_call(
        _mlp_body,
        grid=(g,),
        in_specs=[
            pl.BlockSpec((blk, 2 * _H), lambda i: (i, 0)),
            pl.BlockSpec((blk, 2 * _H), lambda i: (i, 0)),
            pl.BlockSpec((blk, 1), lambda i: (i, 0)),
            pl.BlockSpec((blk, 1), lambda i: (i, 0)),
            pl.BlockSpec((_H, _H), lambda i: (0, 0)),
            pl.BlockSpec((_H, _H), lambda i: (0, 0)),
            pl.BlockSpec((1, _H), lambda i: (0, 0)),
            pl.BlockSpec((1, _H), lambda i: (0, 0)),
            pl.BlockSpec((1, 1), lambda i: (0, 0)),
        ],
        out_specs=pl.BlockSpec((1, 1, blk), lambda i: (i, 0, 0)),
        out_shape=jax.ShapeDtypeStruct((g, 1, blk), jnp.float32),
    )(es, ed, sp, dp, w1a, w1b, b1r, w2r, b2s)
    return out.reshape(_B)


def kernel(mem, W1, b1, W2, b2, s, d):
    s32 = s.astype(jnp.int32)
    d32 = d.astype(jnp.int32)
    mem2 = mem.reshape(mem.shape[0] // 2, 2 * _H)
    es, ed = _gather(mem2, s32, d32)
    sp = jnp.bitwise_and(s32, 1).reshape(_B, 1)
    dp = jnp.bitwise_and(d32, 1).reshape(_B, 1)
    w1a = W1[:_H]
    w1b = W1[_H:]
    b1r = b1.reshape(1, _H)
    w2r = W2.reshape(1, _H)
    b2s = b2.reshape(1, 1)
    return _mlp_tc(es, ed, sp, dp, w1a, w1b, b1r, w2r, b2s)


# speedup vs baseline: 2.4227x; 2.4227x over previous
import functools

import jax
import jax.numpy as jnp
from jax import lax
from jax.experimental import pallas as pl
from jax.experimental.pallas import tpu as pltpu
from jax.experimental.pallas import tpu_sc as plsc

_B = 16384
_H = 64


def _make_gather_kernel():
    info = plsc.get_sparse_core_info()
    nc, ns = info.num_cores, info.num_subcores
    nw = nc * ns
    bpw = _B // nw

    mesh = plsc.VectorSubcoreMesh(core_axis_name="c", subcore_axis_name="s")

    @functools.partial(
        pl.kernel,
        mesh=mesh,
        out_type=[
            jax.ShapeDtypeStruct((_B, _H), jnp.float32),
            jax.ShapeDtypeStruct((_B, _H), jnp.float32),
        ],
        scratch_types=[
            pltpu.VMEM((bpw,), jnp.int32),
            pltpu.VMEM((bpw,), jnp.int32),
            pltpu.VMEM((128, _H), jnp.float32),
            pltpu.VMEM((128, _H), jnp.float32),
            pltpu.VMEM((128, _H), jnp.float32),
            pltpu.VMEM((128, _H), jnp.float32),
            pltpu.SemaphoreType.DMA,
            pltpu.SemaphoreType.DMA,
            pltpu.SemaphoreType.DMA,
            pltpu.SemaphoreType.DMA,
            pltpu.SemaphoreType.DMA,
            pltpu.SemaphoreType.DMA,
            pltpu.SemaphoreType.DMA,
            pltpu.SemaphoreType.DMA,
        ],
    )
    def gk(mem_hbm, s_hbm, d_hbm, es_hbm, ed_hbm,
           sidx_v, didx_v, buf0, buf1, buf2, buf3,
           g0, g1, g2, g3, w0, w1, w2, w3):
        wid = lax.axis_index("s") * nc + lax.axis_index("c")
        base = wid * bpw
        ch = 128
        bufs = (buf0, buf1, buf2, buf3)
        gsems = (g0, g1, g2, g3)
        wsems = (w0, w1, w2, w3)

        pltpu.sync_copy(s_hbm.at[pl.ds(base, bpw)], sidx_v)
        pltpu.sync_copy(d_hbm.at[pl.ds(base, bpw)], didx_v)

        def fire_chunk(idx_ref, off, buf, sem):
            def fire(g, carry):
                j0 = g * 16
                v = idx_ref[pl.ds(off + j0, 16)]
                tv = lax.shift_right_logical(v, 3)
                rv = lax.bitwise_and(v, 7)
                for lane in range(16):
                    pltpu.make_async_copy(
                        mem_hbm.at[tv[lane], pl.ds(rv[lane], 1), :],
                        buf.at[pl.ds(j0 + lane, 1), :], sem).start()
                return carry
            lax.fori_loop(0, ch // 16, fire, 0)

        nch = bpw // ch
        writes = [None] * 4
        for c in range(2 * nch):
            b = c % 4
            if c >= 4:
                writes[b].wait()
            if c < nch:
                idx_ref, off, dst = sidx_v, c * ch, es_hbm
            else:
                idx_ref, off, dst = didx_v, (c - nch) * ch, ed_hbm
            fire_chunk(idx_ref, off, bufs[b], gsems[b])
            pltpu.make_async_copy(
                es_hbm.at[pl.ds(0, ch)], bufs[b], gsems[b]).wait()
            writes[b] = pltpu.async_copy(
                bufs[b], dst.at[pl.ds(base + off, ch)], wsems[b])
        for b in range(4):
            writes[b].wait()

    return gk


_gather = _make_gather_kernel()


def _mlp_body(es_ref, ed_ref, w1a_ref, w1b_ref, b1_ref, w2_ref, b2_ref,
              out_ref):
    x = (jnp.dot(es_ref[...], w1a_ref[...],
                 preferred_element_type=jnp.float32)
         + jnp.dot(ed_ref[...], w1b_ref[...],
                   preferred_element_type=jnp.float32))
    h = jnp.maximum(x + b1_ref[...], 0.0)
    o = jnp.sum(h * w2_ref[...], axis=1) + b2_ref[0, 0]
    out_ref[...] = o.reshape(1, 1, -1)


def _mlp_tc(es, ed, w1a, w1b, b1r, w2r, b2s):
    blk = 2048
    g = _B // blk
    out = pl.pallas_call(
        _mlp_body,
        grid=(g,),
        in_specs=[
            pl.BlockSpec((blk, _H), lambda i: (i, 0)),
            pl.BlockSpec((blk, _H), lambda i: (i, 0)),
            pl.BlockSpec((_H, _H), lambda i: (0, 0)),
            pl.BlockSpec((_H, _H), lambda i: (0, 0)),
            pl.BlockSpec((1, _H), lambda i: (0, 0)),
            pl.BlockSpec((1, _H), lambda i: (0, 0)),
            pl.BlockSpec((1, 1), lambda i: (0, 0)),
        ],
        out_specs=pl.BlockSpec((1, 1, blk), lambda i: (i, 0, 0)),
        out_shape=jax.ShapeDtypeStruct((g, 1, blk), jnp.float32),
    )(es, ed, w1a, w1b, b1r, w2r, b2s)
    return out.reshape(_B)


def kernel(mem, W1, b1, W2, b2, s, d):
    s32 = s.astype(jnp.int32)
    d32 = d.astype(jnp.int32)
    mem3 = mem.reshape(mem.shape[0] // 8, 8, _H)
    es, ed = _gather(mem3, s32, d32)
    w1a = W1[:_H]
    w1b = W1[_H:]
    b1r = b1.reshape(1, _H)
    w2r = W2.reshape(1, _H)
    b2s = b2.reshape(1, 1)
    return _mlp_tc(es, ed, w1a, w1b, b1r, w2r, b2s)


# speedup vs baseline: 2.4965x; 1.0305x over previous
import functools

import jax
import jax.numpy as jnp
from jax import lax
from jax.experimental import pallas as pl
from jax.experimental.pallas import tpu as pltpu
from jax.experimental.pallas import tpu_sc as plsc

_B = 16384
_H = 64


def _make_gather_kernel():
    info = plsc.get_sparse_core_info()
    nc, ns = info.num_cores, info.num_subcores
    nw = nc * ns
    bpw = _B // nw

    mesh = plsc.VectorSubcoreMesh(core_axis_name="c", subcore_axis_name="s")

    @functools.partial(
        pl.kernel,
        mesh=mesh,
        out_type=[
            jax.ShapeDtypeStruct((_B, _H), jnp.float32),
            jax.ShapeDtypeStruct((_B, _H), jnp.float32),
        ],
        scratch_types=[
            pltpu.VMEM((bpw,), jnp.int32),
            pltpu.VMEM((bpw,), jnp.int32),
            pltpu.VMEM((128, _H), jnp.float32),
            pltpu.VMEM((128, _H), jnp.float32),
            pltpu.VMEM((128, _H), jnp.float32),
            pltpu.VMEM((128, _H), jnp.float32),
            pltpu.SemaphoreType.DMA,
            pltpu.SemaphoreType.DMA,
            pltpu.SemaphoreType.DMA,
            pltpu.SemaphoreType.DMA,
            pltpu.SemaphoreType.DMA,
            pltpu.SemaphoreType.DMA,
            pltpu.SemaphoreType.DMA,
            pltpu.SemaphoreType.DMA,
        ],
    )
    def gk(mem_hbm, s_hbm, d_hbm, es_hbm, ed_hbm,
           sidx_v, didx_v, buf0, buf1, buf2, buf3,
           g0, g1, g2, g3, w0, w1, w2, w3):
        wid = lax.axis_index("s") * nc + lax.axis_index("c")
        base = wid * bpw
        ch = 128
        bufs = (buf0, buf1, buf2, buf3)
        gsems = (g0, g1, g2, g3)
        wsems = (w0, w1, w2, w3)

        pltpu.sync_copy(s_hbm.at[pl.ds(base, bpw)], sidx_v)
        pltpu.sync_copy(d_hbm.at[pl.ds(base, bpw)], didx_v)

        def fire_chunk(idx_ref, off, buf, sem):
            def fire(g, carry):
                j0 = g * 16
                v = idx_ref[pl.ds(off + j0, 16)]
                tv = lax.shift_right_logical(v, 3)
                rv = lax.bitwise_and(v, 7)
                for lane in range(16):
                    pltpu.make_async_copy(
                        mem_hbm.at[tv[lane], pl.ds(rv[lane], 1), :],
                        buf.at[pl.ds(j0 + lane, 1), :], sem).start()
                return carry
            lax.fori_loop(0, ch // 16, fire, 0)

        nch = bpw // ch
        writes = [None] * 4

        def drain_and_write(c):
            b = c % 4
            if c < nch:
                off, dst = c * ch, es_hbm
            else:
                off, dst = (c - nch) * ch, ed_hbm
            pltpu.make_async_copy(
                es_hbm.at[pl.ds(0, ch)], bufs[b], gsems[b]).wait()
            writes[b] = pltpu.async_copy(
                bufs[b], dst.at[pl.ds(base + off, ch)], wsems[b])

        for c in range(2 * nch):
            b = c % 4
            if c >= 4:
                writes[b].wait()
            if c < nch:
                idx_ref, off = sidx_v, c * ch
            else:
                idx_ref, off = didx_v, (c - nch) * ch
            fire_chunk(idx_ref, off, bufs[b], gsems[b])
            if c >= 1:
                drain_and_write(c - 1)
        drain_and_write(2 * nch - 1)
        for b in range(4):
            writes[b].wait()

    return gk


_gather = _make_gather_kernel()


def _mlp_body(es_ref, ed_ref, w1a_ref, w1b_ref, b1_ref, w2_ref, b2_ref,
              out_ref):
    x = (jnp.dot(es_ref[...], w1a_ref[...],
                 preferred_element_type=jnp.float32)
         + jnp.dot(ed_ref[...], w1b_ref[...],
                   preferred_element_type=jnp.float32))
    h = jnp.maximum(x + b1_ref[...], 0.0)
    out_ref[...] = (jnp.dot(h, w2_ref[...],
                            preferred_element_type=jnp.float32)
                    + b2_ref[0, 0])


def _mlp_tc(es, ed, w1a, w1b, b1r, w2c, b2s):
    blk = 4096
    g = _B // blk
    out = pl.pallas_call(
        _mlp_body,
        grid=(g,),
        in_specs=[
            pl.BlockSpec((blk, _H), lambda i: (i, 0)),
            pl.BlockSpec((blk, _H), lambda i: (i, 0)),
            pl.BlockSpec((_H, _H), lambda i: (0, 0)),
            pl.BlockSpec((_H, _H), lambda i: (0, 0)),
            pl.BlockSpec((1, _H), lambda i: (0, 0)),
            pl.BlockSpec((_H, 1), lambda i: (0, 0)),
            pl.BlockSpec((1, 1), lambda i: (0, 0)),
        ],
        out_specs=pl.BlockSpec((blk, 1), lambda i: (i, 0)),
        out_shape=jax.ShapeDtypeStruct((_B, 1), jnp.float32),
    )(es, ed, w1a, w1b, b1r, w2c, b2s)
    return out.reshape(_B)


def kernel(mem, W1, b1, W2, b2, s, d):
    s32 = s.astype(jnp.int32)
    d32 = d.astype(jnp.int32)
    mem3 = mem.reshape(mem.shape[0] // 8, 8, _H)
    es, ed = _gather(mem3, s32, d32)
    w1a = W1[:_H]
    w1b = W1[_H:]
    b1r = b1.reshape(1, _H)
    b2s = b2.reshape(1, 1)
    return _mlp_tc(es, ed, w1a, w1b, b1r, W2, b2s)


# speedup vs baseline: 2.5015x; 1.0020x over previous
import functools

import jax
import jax.numpy as jnp
from jax import lax
from jax.experimental import pallas as pl
from jax.experimental.pallas import tpu as pltpu
from jax.experimental.pallas import tpu_sc as plsc

_B = 16384
_H = 64


def _make_gather_kernel():
    info = plsc.get_sparse_core_info()
    nc, ns = info.num_cores, info.num_subcores
    nw = nc * ns
    bpw = _B // nw

    mesh = plsc.VectorSubcoreMesh(core_axis_name="c", subcore_axis_name="s")

    @functools.partial(
        pl.kernel,
        mesh=mesh,
        out_type=[
            jax.ShapeDtypeStruct((_B, _H), jnp.float32),
            jax.ShapeDtypeStruct((_B, _H), jnp.float32),
        ],
        scratch_types=[
            pltpu.VMEM((bpw,), jnp.int32),
            pltpu.VMEM((bpw,), jnp.int32),
            pltpu.VMEM((128, _H), jnp.float32),
            pltpu.VMEM((128, _H), jnp.float32),
            pltpu.VMEM((128, _H), jnp.float32),
            pltpu.VMEM((128, _H), jnp.float32),
            pltpu.SemaphoreType.DMA,
            pltpu.SemaphoreType.DMA,
            pltpu.SemaphoreType.DMA,
            pltpu.SemaphoreType.DMA,
            pltpu.SemaphoreType.DMA,
            pltpu.SemaphoreType.DMA,
            pltpu.SemaphoreType.DMA,
            pltpu.SemaphoreType.DMA,
        ],
    )
    def gk(mem_hbm, s_hbm, d_hbm, es_hbm, ed_hbm,
           sidx_v, didx_v, buf0, buf1, buf2, buf3,
           g0, g1, g2, g3, w0, w1, w2, w3):
        wid = lax.axis_index("s") * nc + lax.axis_index("c")
        base = wid * bpw
        ch = 128
        bufs = (buf0, buf1, buf2, buf3)
        gsems = (g0, g1, g2, g3)
        wsems = (w0, w1, w2, w3)

        pltpu.sync_copy(s_hbm.at[pl.ds(base, bpw)], sidx_v)
        pltpu.sync_copy(d_hbm.at[pl.ds(base, bpw)], didx_v)

        def fire_chunk(idx_ref, off, buf, sem):
            def fire(g, carry):
                j0 = g * 16
                v = idx_ref[pl.ds(off + j0, 16)]
                tv = lax.shift_right_logical(v, 3)
                rv = lax.bitwise_and(v, 7)
                for lane in range(16):
                    pltpu.make_async_copy(
                        mem_hbm.at[tv[lane], pl.ds(rv[lane], 1), :],
                        buf.at[pl.ds(j0 + lane, 1), :], sem).start()
                return carry
            lax.fori_loop(0, ch // 16, fire, 0)

        nch = bpw // ch
        writes = [None] * 4

        def drain_and_write(c):
            b = c % 4
            if c < nch:
                off, dst = c * ch, es_hbm
            else:
                off, dst = (c - nch) * ch, ed_hbm
            pltpu.make_async_copy(
                es_hbm.at[pl.ds(0, ch)], bufs[b], gsems[b]).wait()
            writes[b] = pltpu.async_copy(
                bufs[b], dst.at[pl.ds(base + off, ch)], wsems[b])

        for c in range(2 * nch):
            b = c % 4
            if c >= 4:
                writes[b].wait()
            if c < nch:
                idx_ref, off = sidx_v, c * ch
            else:
                idx_ref, off = didx_v, (c - nch) * ch
            fire_chunk(idx_ref, off, bufs[b], gsems[b])
            if c >= 1:
                drain_and_write(c - 1)
        drain_and_write(2 * nch - 1)
        for b in range(4):
            writes[b].wait()

    return gk


_gather = _make_gather_kernel()


def _mlp_body(es_ref, ed_ref, w1a_ref, w1b_ref, b1_ref, w2_ref, b2_ref,
              out_ref):
    x = (jnp.dot(es_ref[...], w1a_ref[...],
                 preferred_element_type=jnp.float32)
         + jnp.dot(ed_ref[...], w1b_ref[...],
                   preferred_element_type=jnp.float32))
    h = jnp.maximum(x + b1_ref[...], 0.0)
    out_ref[...] = (jnp.dot(h, w2_ref[...],
                            preferred_element_type=jnp.float32)
                    + b2_ref[0, 0])


def _mlp_tc(es, ed, w1a, w1b, b1r, w2c, b2s):
    blk = 8192
    g = _B // blk
    out = pl.pallas_call(
        _mlp_body,
        grid=(g,),
        in_specs=[
            pl.BlockSpec((blk, _H), lambda i: (i, 0)),
            pl.BlockSpec((blk, _H), lambda i: (i, 0)),
            pl.BlockSpec((_H, _H), lambda i: (0, 0)),
            pl.BlockSpec((_H, _H), lambda i: (0, 0)),
            pl.BlockSpec((1, _H), lambda i: (0, 0)),
            pl.BlockSpec((_H, 1), lambda i: (0, 0)),
            pl.BlockSpec((1, 1), lambda i: (0, 0)),
        ],
        out_specs=pl.BlockSpec((blk, 1), lambda i: (i, 0)),
        out_shape=jax.ShapeDtypeStruct((_B, 1), jnp.float32),
    )(es, ed, w1a, w1b, b1r, w2c, b2s)
    return out.reshape(_B)


def kernel(mem, W1, b1, W2, b2, s, d):
    s32 = s.astype(jnp.int32)
    d32 = d.astype(jnp.int32)
    mem3 = mem.reshape(mem.shape[0] // 8, 8, _H)
    es, ed = _gather(mem3, s32, d32)
    w1a = W1[:_H]
    w1b = W1[_H:]
    b1r = b1.reshape(1, _H)
    b2s = b2.reshape(1, 1)
    return _mlp_tc(es, ed, w1a, w1b, b1r, W2, b2s)
